# Initial kernel scaffold; baseline (speedup 1.0000x reference)
import jax
import jax.numpy as jnp
from jax.experimental import pallas as pl


def kernel(input_spikes, max_timesteps, weights, targets):
    raise NotImplementedError("write your pallas kernel here")


# speedup vs baseline: 54.8341x; 54.8341x over previous
import functools
import math

import jax
import jax.numpy as jnp
from jax import lax
from jax.experimental import pallas as pl
from jax.experimental.pallas import tpu as pltpu
from jax.experimental.pallas import tpu_sc as plsc

NUM_INPUT = 2048
NUM_HIDDEN = 32768
NUM_OUTPUT = 512
N = NUM_INPUT + NUM_HIDDEN + NUM_OUTPUT
FAN_OUT = 32
TAU = 20.0
THRESHOLD = 0.3
HID_START = NUM_INPUT
OUT_START = NUM_INPUT + NUM_HIDDEN
CHK = N - HID_START
OUT_CHK = OUT_START - HID_START
MAX_STEPS = 20
CHUNK = 128
L = 16

_INV_DECAY = float(math.exp(1.0 / TAU))


def _snn_body(spk_hbm, combo_hbm, times_hbm, pots_hbm,
              p_v, fired_v, front_v, spk_v, idx_v, rows_v,
              times_v, sem_a):
    wid = lax.axis_index("s") * 2 + lax.axis_index("c")

    @pl.when(wid == 0)
    def _():
        def init_body(i, _):
            sl = pl.ds(i * L, L)
            p_v[sl] = jnp.zeros((L,), jnp.float32)
            fired_v[sl] = jnp.zeros((L,), jnp.int32)
            front_v[sl] = jnp.zeros((L,), jnp.int32)
            return 0
        lax.fori_loop(0, CHK // L, init_body, 0)

        def tinit_body(i, _):
            times_v[pl.ds(i * L, L)] = jnp.full((L,), -1, jnp.int32)
            return 0
        lax.fori_loop(0, NUM_OUTPUT // L, tinit_body, 0)

        pltpu.sync_copy(spk_hbm, spk_v)

        def in_body(i, off):
            m = spk_v[pl.ds(i * L, L)] > 0
            ids = jax.lax.iota(jnp.int32, L) + i * L
            mi = jnp.where(m, jnp.int32(1), jnp.int32(0))
            pos = off + plsc.cumsum(mi) - 1
            plsc.store_scatter(front_v, [pos], ids, mask=m)
            return off + jnp.sum(mi)
        cnt0 = lax.fori_loop(0, NUM_INPUT // L, in_body, jnp.int32(0))

        def step_body(t, carry):
            cnt, g = carry
            amp = jnp.where(t == 0, jnp.float32(2.0) * g, g)
            thr = jnp.float32(THRESHOLD) * g

            def chunk_body(c, _):
                base = c * CHUNK

                def cp_body(j, _):
                    idx_v[pl.ds(j * L, L)] = front_v[pl.ds(base + j * L, L)]
                    return 0
                lax.fori_loop(0, CHUNK // L, cp_body, 0)
                pltpu.async_copy(combo_hbm.at[idx_v], rows_v, sem_a).wait()
                nrows = jnp.minimum(jnp.int32(CHUNK), cnt - base)

                def row_body(r, _):
                    for h in range(FAN_OUT // L):
                        tv = rows_v[r, pl.ds(h * L, L)] - HID_START
                        wb = rows_v[r, pl.ds(FAN_OUT + h * L, L)]
                        wv = plsc.bitcast(wb, jnp.float32) * amp
                        plsc.addupdate_scatter(p_v, [tv], wv)
                    return 0
                lax.fori_loop(0, nrows, row_body, 0)
                return 0
            nchunks = (cnt + (CHUNK - 1)) // CHUNK
            lax.fori_loop(0, nchunks, chunk_body, 0)

            def hid_body(i, off):
                sl = pl.ds(i * L, L)
                v = p_v[sl]
                f = fired_v[sl]
                m = (v >= thr) & (f == 0)
                mi = jnp.where(m, jnp.int32(1), jnp.int32(0))
                fired_v[sl] = f | mi
                p_v[sl] = jnp.where(m, jnp.float32(0.0), v)
                ids = jax.lax.iota(jnp.int32, L) + (HID_START + i * L)
                pos = off + plsc.cumsum(mi) - 1
                plsc.store_scatter(front_v, [pos], ids, mask=m)
                return off + jnp.sum(mi)

            def out_body(i, off):
                sl = pl.ds(OUT_CHK + i * L, L)
                v = p_v[sl]
                f = fired_v[sl]
                m = (v >= thr) & (f == 0)
                mi = jnp.where(m, jnp.int32(1), jnp.int32(0))
                fired_v[sl] = f | mi
                tsl = pl.ds(i * L, L)
                times_v[tsl] = jnp.where(m, t, times_v[tsl])
                ids = jax.lax.iota(jnp.int32, L) + (OUT_START + i * L)
                pos = off + plsc.cumsum(mi) - 1
                plsc.store_scatter(front_v, [pos], ids, mask=m)
                return off + jnp.sum(mi)

            hid_trips = jnp.where(cnt > 0, OUT_CHK // L, 0)
            out_trips = jnp.where(cnt > 0, NUM_OUTPUT // L, 0)
            off = lax.fori_loop(0, hid_trips, hid_body, jnp.int32(0))
            off = lax.fori_loop(0, out_trips, out_body, off)
            return off, g * jnp.float32(_INV_DECAY)

        lax.fori_loop(0, MAX_STEPS, step_body, (cnt0, jnp.float32(1.0)))

        pltpu.sync_copy(times_v, times_hbm)
        pltpu.sync_copy(p_v.at[pl.ds(OUT_CHK, NUM_OUTPUT)], pots_hbm)


_snn = pl.kernel(
    _snn_body,
    out_type=(jax.ShapeDtypeStruct((NUM_OUTPUT,), jnp.int32),
              jax.ShapeDtypeStruct((NUM_OUTPUT,), jnp.float32)),
    mesh=plsc.VectorSubcoreMesh(core_axis_name="c", subcore_axis_name="s"),
    compiler_params=pltpu.CompilerParams(needs_layout_passes=False),
    scratch_types=[
        pltpu.VMEM((CHK,), jnp.float32),
        pltpu.VMEM((CHK,), jnp.int32),
        pltpu.VMEM((CHK,), jnp.int32),
        pltpu.VMEM((NUM_INPUT,), jnp.int32),
        pltpu.VMEM((CHUNK,), jnp.int32),
        pltpu.VMEM((CHUNK, 128), jnp.int32),
        pltpu.VMEM((NUM_OUTPUT,), jnp.int32),
        pltpu.SemaphoreType.DMA,
    ],
)


def kernel(input_spikes, max_timesteps, weights, targets):
    spk = input_spikes.astype(jnp.int32)
    combo = jnp.concatenate(
        [targets,
         jax.lax.bitcast_convert_type(weights, jnp.int32),
         jnp.zeros((N, 128 - 2 * FAN_OUT), jnp.int32)], axis=1)
    times, pots_scaled = _snn(spk, combo)
    decay_base = jnp.exp(jnp.array(-1.0 / TAU, dtype=jnp.float32))
    scale = decay_base ** jnp.asarray(max_timesteps, jnp.float32)
    return times, pots_scaled * scale


# speedup vs baseline: 69.6385x; 1.2700x over previous
import math

import jax
import jax.numpy as jnp
from jax import lax
from jax.experimental import pallas as pl
from jax.experimental.pallas import tpu as pltpu
from jax.experimental.pallas import tpu_sc as plsc

NUM_INPUT = 2048
NUM_HIDDEN = 32768
NUM_OUTPUT = 512
N = NUM_INPUT + NUM_HIDDEN + NUM_OUTPUT
FAN_OUT = 32
TAU = 20.0
THRESHOLD = 0.3
HID_START = NUM_INPUT
OUT_START = NUM_INPUT + NUM_HIDDEN
CHK = N - HID_START
OUT_CHK = OUT_START - HID_START
MAX_STEPS = 20
CHUNK = 128
L = 16
ROW = 128
NEG = -1.0e30

_INV_DECAY = float(math.exp(1.0 / TAU))


def _snn_body(spk_hbm, combo_hbm, times_hbm, pots_hbm,
              p_v, front_v, spk_v, fired_o, times_v,
              idx0, idx1, rows0, rows1, sem0, sem1):
    wid = lax.axis_index("s") * 2 + lax.axis_index("c")

    @pl.when(wid == 0)
    def _():
        def init_body(i, _):
            sl = pl.ds(i * L, L)
            p_v[sl] = jnp.zeros((L,), jnp.float32)
            front_v[sl] = jnp.zeros((L,), jnp.int32)
            return 0
        lax.fori_loop(0, CHK // L, init_body, 0)

        def tinit_body(i, _):
            times_v[pl.ds(i * L, L)] = jnp.full((L,), -1, jnp.int32)
            fired_o[pl.ds(i * L, L)] = jnp.zeros((L,), jnp.int32)
            return 0
        lax.fori_loop(0, NUM_OUTPUT // L, tinit_body, 0)

        pltpu.sync_copy(spk_hbm, spk_v)

        def in_body(i, off):
            m = spk_v[pl.ds(i * L, L)] > 0
            ids = jax.lax.iota(jnp.int32, L) + i * L
            mi = jnp.where(m, jnp.int32(1), jnp.int32(0))
            cs = plsc.cumsum(mi)
            plsc.store_scatter(front_v, [off + cs - 1], ids, mask=m)
            return off + cs[L - 1]
        cnt0 = lax.fori_loop(0, NUM_INPUT // L, in_body, jnp.int32(0))

        def fill_idx(idx_ref, cnt, base):
            b = jnp.minimum(base, jnp.maximum(cnt - 1, 0))
            b = jnp.minimum(b, jnp.int32(CHK - CHUNK))

            def cp_body(j, _):
                idx_ref[pl.ds(j * L, L)] = front_v[pl.ds(b + j * L, L)]
                return 0
            lax.fori_loop(0, CHUNK // L, cp_body, 0)

        def scat_rows(rows_ref, cnt, base, amp):
            nrows = jnp.maximum(
                jnp.minimum(jnp.int32(CHUNK), cnt - base), jnp.int32(0))

            def row_body(r, _):
                for h in range(FAN_OUT // L):
                    tv = rows_ref[r, pl.ds(h * L, L)] - HID_START
                    wb = rows_ref[r, pl.ds(FAN_OUT + h * L, L)]
                    wv = plsc.bitcast(wb, jnp.float32) * amp
                    plsc.addupdate_scatter(p_v, [tv], wv)
                return 0
            lax.fori_loop(0, nrows, row_body, 0)

        def step_cond(carry):
            t, cnt, g = carry
            return (t < MAX_STEPS) & (cnt > 0)

        def step_body(carry):
            t, cnt, g = carry
            amp = jnp.where(t == 0, jnp.float32(2.0) * g, g)
            thr = jnp.float32(THRESHOLD) * g

            nchunks = (cnt + (CHUNK - 1)) // CHUNK
            nhalf = (nchunks + 1) // 2
            fill_idx(idx0, cnt, jnp.int32(0))
            cp = pltpu.async_copy(combo_hbm.at[idx0], rows0, sem0)

            def pair_body(k, _):
                b0 = (2 * k) * CHUNK
                b1 = b0 + CHUNK
                b2 = b1 + CHUNK
                pltpu.make_async_copy(combo_hbm.at[idx0], rows0, sem0).wait()
                fill_idx(idx1, cnt, b1)
                pltpu.async_copy(combo_hbm.at[idx1], rows1, sem1)
                scat_rows(rows0, cnt, b0, amp)
                pltpu.make_async_copy(combo_hbm.at[idx1], rows1, sem1).wait()
                fill_idx(idx0, cnt, b2)
                pltpu.async_copy(combo_hbm.at[idx0], rows0, sem0)
                scat_rows(rows1, cnt, b1, amp)
                return 0
            lax.fori_loop(0, nhalf, pair_body, 0)
            pltpu.make_async_copy(combo_hbm.at[idx0], rows0, sem0).wait()

            UNR = 4

            def hid_body(i, off):
                for u in range(UNR):
                    sl = pl.ds((i * UNR + u) * L, L)
                    v = p_v[sl]
                    m = v >= thr
                    p_v[sl] = jnp.where(m, jnp.float32(NEG), v)
                    ids = (jax.lax.iota(jnp.int32, L)
                           + (HID_START + (i * UNR + u) * L))
                    mi = jnp.where(m, jnp.int32(1), jnp.int32(0))
                    cs = plsc.cumsum(mi)
                    plsc.store_scatter(front_v, [off + cs - 1], ids, mask=m)
                    off = off + cs[L - 1]
                return off

            def out_body(i, off):
                sl = pl.ds(OUT_CHK + i * L, L)
                v = p_v[sl]
                f = fired_o[pl.ds(i * L, L)]
                m = (v >= thr) & (f == 0)
                mi = jnp.where(m, jnp.int32(1), jnp.int32(0))
                fired_o[pl.ds(i * L, L)] = f | mi
                tsl = pl.ds(i * L, L)
                times_v[tsl] = jnp.where(m, t, times_v[tsl])
                ids = jax.lax.iota(jnp.int32, L) + (OUT_START + i * L)
                cs = plsc.cumsum(mi)
                plsc.store_scatter(front_v, [off + cs - 1], ids, mask=m)
                return off + cs[L - 1]

            off = lax.fori_loop(0, OUT_CHK // L // UNR, hid_body, jnp.int32(0))
            off = lax.fori_loop(0, NUM_OUTPUT // L, out_body, off)
            return t + 1, off, g * jnp.float32(_INV_DECAY)

        lax.while_loop(step_cond, step_body,
                       (jnp.int32(0), cnt0, jnp.float32(1.0)))

        pltpu.sync_copy(times_v, times_hbm)
        pltpu.sync_copy(p_v.at[pl.ds(OUT_CHK, NUM_OUTPUT)], pots_hbm)


_snn = pl.kernel(
    _snn_body,
    out_type=(jax.ShapeDtypeStruct((NUM_OUTPUT,), jnp.int32),
              jax.ShapeDtypeStruct((NUM_OUTPUT,), jnp.float32)),
    mesh=plsc.VectorSubcoreMesh(core_axis_name="c", subcore_axis_name="s"),
    compiler_params=pltpu.CompilerParams(needs_layout_passes=False),
    scratch_types=[
        pltpu.VMEM((CHK,), jnp.float32),
        pltpu.VMEM((CHK,), jnp.int32),
        pltpu.VMEM((NUM_INPUT,), jnp.int32),
        pltpu.VMEM((NUM_OUTPUT,), jnp.int32),
        pltpu.VMEM((NUM_OUTPUT,), jnp.int32),
        pltpu.VMEM((CHUNK,), jnp.int32),
        pltpu.VMEM((CHUNK,), jnp.int32),
        pltpu.VMEM((CHUNK, ROW), jnp.int32),
        pltpu.VMEM((CHUNK, ROW), jnp.int32),
        pltpu.SemaphoreType.DMA,
        pltpu.SemaphoreType.DMA,
    ],
)


def kernel(input_spikes, max_timesteps, weights, targets):
    spk = input_spikes.astype(jnp.int32)
    combo = jnp.concatenate(
        [targets,
         jax.lax.bitcast_convert_type(weights, jnp.int32),
         jnp.zeros((N, ROW - 2 * FAN_OUT), jnp.int32)], axis=1)
    times, pots_scaled = _snn(spk, combo)
    decay_base = jnp.exp(jnp.array(-1.0 / TAU, dtype=jnp.float32))
    scale = decay_base ** jnp.asarray(max_timesteps, jnp.float32)
    return times, pots_scaled * scale


# speedup vs baseline: 189.8430x; 2.7261x over previous
import math

import jax
import jax.numpy as jnp
from jax import lax
from jax.experimental import pallas as pl
from jax.experimental.pallas import tpu as pltpu
from jax.experimental.pallas import tpu_sc as plsc

NUM_INPUT = 2048
NUM_HIDDEN = 32768
NUM_OUTPUT = 512
N = NUM_INPUT + NUM_HIDDEN + NUM_OUTPUT
FAN_OUT = 32
TAU = 20.0
THRESHOLD = 0.3
HID_START = NUM_INPUT
OUT_START = NUM_INPUT + NUM_HIDDEN
CHK = N - HID_START
OUT_CHK = OUT_START - HID_START
MAX_STEPS = 20
L = 16
NSH = 16
SHARD = CHK // NSH
SH_VECS = SHARD // L
ECH = 128
ROW = 128
NSTR = ECH * FAN_OUT // 128
IN_PER = NUM_INPUT // NSH

_INV_DECAY = float(math.exp(1.0 / TAU))


def _snn_body(spk_hbm, combo_hbm, times_hbm, pots_hbm,
              shard_p, fired_f, front_v, spk_v, idx_v, rows_v,
              eidx_v, eval_v, times_v, cnt_v, counts_all,
              p_sh, counts_sh, sem_g, sem_s):
    cid = lax.axis_index("c")
    sid = lax.axis_index("s")

    @pl.when(cid == 0)
    def _():
        my_base = sid * SHARD

        def init_body(i, _):
            sl = pl.ds(i * L, L)
            shard_p[sl] = jnp.zeros((L,), jnp.float32)
            fired_f[sl] = jnp.zeros((L,), jnp.int32)
            front_v[sl] = jnp.zeros((L,), jnp.int32)
            return 0
        lax.fori_loop(0, SH_VECS, init_body, 0)

        def tinit_body(i, _):
            times_v[pl.ds(i * L, L)] = jnp.full((L,), -1, jnp.int32)
            return 0
        lax.fori_loop(0, NUM_OUTPUT // L, tinit_body, 0)

        pltpu.sync_copy(shard_p, p_sh.at[pl.ds(my_base, SHARD)])

        pltpu.sync_copy(spk_hbm.at[pl.ds(sid * IN_PER, IN_PER)], spk_v)

        def in_body(i, off):
            m = spk_v[pl.ds(i * L, L)] > 0
            ids = jax.lax.iota(jnp.int32, L) + (sid * IN_PER + i * L)
            mi = jnp.where(m, jnp.int32(1), jnp.int32(0))
            cs = plsc.cumsum(mi)
            plsc.store_scatter(front_v, [off + cs - 1], ids, mask=m)
            return off + cs[L - 1]
        cnt0 = lax.fori_loop(0, IN_PER // L, in_body, jnp.int32(0))

        def publish_total(my_cnt):
            cnt_v[pl.ds(0, L)] = jnp.full((L,), my_cnt, jnp.int32)
            pltpu.sync_copy(cnt_v, counts_sh.at[sid])
            plsc.subcore_barrier()
            pltpu.sync_copy(counts_sh, counts_all)
            tot = jnp.zeros((L,), jnp.int32)
            for j in range(NSH):
                tot = tot + counts_all[j, pl.ds(0, L)]
            return tot[0]

        tot0 = publish_total(cnt0)

        def step_cond(carry):
            t, my_cnt, total, g = carry
            return (t < MAX_STEPS) & (total > 0)

        def step_body(carry):
            t, my_cnt, total, g = carry
            amp = jnp.where(t == 0, jnp.float32(2.0) * g, g)
            thr = jnp.float32(THRESHOLD) * g

            def chunk_body(c, _):
                base = c * ECH
                b = jnp.minimum(base, jnp.maximum(my_cnt - 1, 0))
                b = jnp.minimum(b, jnp.int32(SHARD - ECH))

                def cp_body(j, _):
                    idx_v[pl.ds(j * L, L)] = front_v[pl.ds(b + j * L, L)]
                    return 0
                lax.fori_loop(0, ECH // L, cp_body, 0)
                pltpu.async_copy(combo_hbm.at[idx_v], rows_v, sem_g).wait()

                nrows = jnp.maximum(
                    jnp.minimum(jnp.int32(ECH), my_cnt - base), jnp.int32(0))
                rup = ((nrows + 3) // 4) * 4

                def row_body(r, _):
                    valid = r < nrows
                    j = r // 4
                    col = (r % 4) * (2 * L)
                    for h in range(FAN_OUT // L):
                        tv = rows_v[r, pl.ds(h * L, L)] - HID_START
                        wb = rows_v[r, pl.ds(FAN_OUT + h * L, L)]
                        wv = plsc.bitcast(wb, jnp.float32) * amp
                        tv = jnp.where(valid, tv, jnp.int32(0))
                        wv = jnp.where(valid, wv, jnp.float32(0.0))
                        eidx_v[j, pl.ds(col + h * L, L)] = tv
                        eval_v[j, pl.ds(col + h * L, L)] = wv
                    return 0
                lax.fori_loop(0, rup, row_body, 0)

                for j in range(NSTR):
                    @pl.when(j * 4 < rup)
                    def _(j=j):
                        pltpu.async_copy(
                            eval_v.at[j], p_sh.at[eidx_v.at[j]], sem_s,
                            add=True)
                for j in range(NSTR):
                    @pl.when(j * 4 < rup)
                    def _(j=j):
                        pltpu.make_async_copy(
                            eval_v.at[j], p_sh.at[eidx_v.at[j]], sem_s).wait()
                return 0
            nchunks = (my_cnt + (ECH - 1)) // ECH
            lax.fori_loop(0, nchunks, chunk_body, 0)
            plsc.subcore_barrier()

            pltpu.sync_copy(p_sh.at[pl.ds(my_base, SHARD)], shard_p)

            def sw_body(i, off):
                co = my_base + i * L
                sl = pl.ds(i * L, L)
                v = shard_p[sl]
                f = fired_f[sl]
                m = (v >= thr) & (f == 0)
                mi = jnp.where(m, jnp.int32(1), jnp.int32(0))
                fired_f[sl] = f | mi
                keep = m & (co < OUT_CHK)
                shard_p[sl] = jnp.where(keep, jnp.float32(0.0), v)

                @pl.when(co >= OUT_CHK)
                def _():
                    osl = pl.ds(co - OUT_CHK, L)
                    times_v[osl] = jnp.where(m, t, times_v[osl])

                ids = jax.lax.iota(jnp.int32, L) + (co + HID_START)
                cs = plsc.cumsum(mi)
                plsc.store_scatter(front_v, [off + cs - 1], ids, mask=m)
                return off + cs[L - 1]
            my_new = lax.fori_loop(0, SH_VECS, sw_body, jnp.int32(0))

            pltpu.sync_copy(shard_p, p_sh.at[pl.ds(my_base, SHARD)])
            new_total = publish_total(my_new)
            return t + 1, my_new, new_total, g * jnp.float32(_INV_DECAY)

        lax.while_loop(step_cond, step_body,
                       (jnp.int32(0), cnt0, tot0, jnp.float32(1.0)))

        @pl.when(sid == NSH - 1)
        def _():
            pltpu.sync_copy(times_v, times_hbm)
            pltpu.sync_copy(
                shard_p.at[pl.ds(OUT_CHK - (NSH - 1) * SHARD, NUM_OUTPUT)],
                pots_hbm)


_snn = pl.kernel(
    _snn_body,
    out_type=(jax.ShapeDtypeStruct((NUM_OUTPUT,), jnp.int32),
              jax.ShapeDtypeStruct((NUM_OUTPUT,), jnp.float32)),
    mesh=plsc.VectorSubcoreMesh(core_axis_name="c", subcore_axis_name="s"),
    compiler_params=pltpu.CompilerParams(needs_layout_passes=False),
    scratch_types=[
        pltpu.VMEM((SHARD,), jnp.float32),
        pltpu.VMEM((SHARD,), jnp.int32),
        pltpu.VMEM((SHARD,), jnp.int32),
        pltpu.VMEM((IN_PER,), jnp.int32),
        pltpu.VMEM((ECH,), jnp.int32),
        pltpu.VMEM((ECH, ROW), jnp.int32),
        pltpu.VMEM((NSTR, 128), jnp.int32),
        pltpu.VMEM((NSTR, 128), jnp.float32),
        pltpu.VMEM((NUM_OUTPUT,), jnp.int32),
        pltpu.VMEM((L,), jnp.int32),
        pltpu.VMEM((NSH, L), jnp.int32),
        pltpu.VMEM_SHARED((CHK,), jnp.float32),
        pltpu.VMEM_SHARED((NSH, L), jnp.int32),
        pltpu.SemaphoreType.DMA,
        pltpu.SemaphoreType.DMA,
    ],
)


def kernel(input_spikes, max_timesteps, weights, targets):
    spk = input_spikes.astype(jnp.int32)
    combo = jnp.concatenate(
        [targets,
         jax.lax.bitcast_convert_type(weights, jnp.int32),
         jnp.zeros((N, ROW - 2 * FAN_OUT), jnp.int32)], axis=1)
    times, pots_scaled = _snn(spk, combo)
    decay_base = jnp.exp(jnp.array(-1.0 / TAU, dtype=jnp.float32))
    scale = decay_base ** jnp.asarray(max_timesteps, jnp.float32)
    return times, pots_scaled * scale
